# baseline (speedup 1.0000x reference)
import functools

import jax
import jax.numpy as jnp
from jax import lax
from jax.experimental import pallas as pl
from jax.experimental.pallas import tpu as pltpu
from jax.experimental.pallas import tpu_sc as plsc

B = 2
TSEQ = 16
N = 10000
IN_CH = 3
HID = 64
OUT_CH = 16
KS = 3
NCLS = 10

NT = 2000
NG = N // NT

INTERPRET = False


def _tconv1_call(x, wcat, bcat, deg):
    Bb, T, _, cin = x.shape
    To = T - 2
    S = Bb * To

    def body(x0, x1, x2, w, b, dg, out, outs):
        y = (jnp.dot(x0[0, 0], w[0], preferred_element_type=jnp.float32)
             + jnp.dot(x1[0, 0], w[1], preferred_element_type=jnp.float32)
             + jnp.dot(x2[0, 0], w[2], preferred_element_type=jnp.float32)
             + b[0])
        g = jax.nn.relu(y[:, :HID] * jax.nn.sigmoid(y[:, HID:2 * HID])
                        + y[:, 2 * HID:])
        dis = jnp.where(dg[...] > 0, lax.rsqrt(dg[...]), 0.0)
        out[0] = g
        outs[0] = g * dis

    xspec = lambda k: pl.BlockSpec((1, 1, NT, cin), lambda bi, t, n: (bi, t + k, n, 0))
    return pl.pallas_call(
        body,
        grid=(Bb, To, NG),
        in_specs=[
            xspec(0), xspec(1), xspec(2),
            pl.BlockSpec((KS, cin, 3 * HID), lambda bi, t, n: (0, 0, 0)),
            pl.BlockSpec((1, 3 * HID), lambda bi, t, n: (0, 0)),
            pl.BlockSpec((NT, 1), lambda bi, t, n: (n, 0)),
        ],
        out_specs=[
            pl.BlockSpec((1, NT, HID), lambda bi, t, n: (bi * To + t, n, 0)),
            pl.BlockSpec((1, NT, HID), lambda bi, t, n: (bi * To + t, n, 0)),
        ],
        out_shape=[
            jax.ShapeDtypeStruct((S, N, HID), jnp.float32),
            jax.ShapeDtypeStruct((S, N, HID), jnp.float32),
        ],
        interpret=INTERPRET,
    )(x, x, x, wcat, bcat, deg)


def _cheb_combine_call(xc, g1, g2, deg, chw, chb):
    S = xc.shape[0]

    def body(xr, g1r, g2r, dg, w, b, out):
        dis = jnp.where(dg[...] > 0, lax.rsqrt(dg[...]), 0.0)
        x = xr[0]
        a = g1r[0] * (-dis)
        c = g2r[0] * (2.0 * dis)
        z = (jnp.dot(x, w[0] - w[2], preferred_element_type=jnp.float32)
             + jnp.dot(a, w[1], preferred_element_type=jnp.float32)
             + jnp.dot(c, w[2], preferred_element_type=jnp.float32)
             + b[0])
        out[0] = jax.nn.relu(z)

    fspec = pl.BlockSpec((1, NT, HID), lambda s, n: (s, n, 0))
    return pl.pallas_call(
        body,
        grid=(S, NG),
        in_specs=[
            fspec, fspec, fspec,
            pl.BlockSpec((NT, 1), lambda s, n: (n, 0)),
            pl.BlockSpec((3, HID, HID), lambda s, n: (0, 0, 0)),
            pl.BlockSpec((1, HID), lambda s, n: (0, 0)),
        ],
        out_specs=fspec,
        out_shape=jax.ShapeDtypeStruct((S, N, HID), jnp.float32),
        interpret=INTERPRET,
    )(xc, g1, g2, deg, chw, chb.reshape(1, HID))


def _scale_call(g1, deg):
    S = g1.shape[0]

    def body(g1r, dg, out):
        inv = jnp.where(dg[...] > 0, 1.0 / dg[...], 0.0)
        out[0] = g1r[0] * inv

    fspec = pl.BlockSpec((1, NT, HID), lambda s, n: (s, n, 0))
    return pl.pallas_call(
        body,
        grid=(S, NG),
        in_specs=[fspec, pl.BlockSpec((NT, 1), lambda s, n: (n, 0))],
        out_specs=fspec,
        out_shape=jax.ShapeDtypeStruct((S, N, HID), jnp.float32),
        interpret=INTERPRET,
    )(g1, deg)


def _tconv2_call(z, wcat, bcat, bn):
    S_in = z.shape[0]
    T = S_in // B
    To = T - 2
    zr = z.reshape(B, T, N, HID)

    def body(x0, x1, x2, w, b, bnr, out):
        y = (jnp.dot(x0[0, 0], w[0], preferred_element_type=jnp.float32)
             + jnp.dot(x1[0, 0], w[1], preferred_element_type=jnp.float32)
             + jnp.dot(x2[0, 0], w[2], preferred_element_type=jnp.float32)
             + b[0])
        g = jax.nn.relu(y[:, :OUT_CH] * jax.nn.sigmoid(y[:, OUT_CH:2 * OUT_CH])
                        + y[:, 2 * OUT_CH:])
        gam, bet, m, v = bnr[:, 0:1], bnr[:, 1:2], bnr[:, 2:3], bnr[:, 3:4]
        scale = gam * lax.rsqrt(v + 1e-5)
        shift = bet - m * scale
        out[0, 0] = jax.nn.relu(g * scale + shift)

    xspec = lambda k: pl.BlockSpec((1, 1, NT, HID), lambda bi, t, n: (bi, t + k, n, 0))
    return pl.pallas_call(
        body,
        grid=(B, To, NG),
        in_specs=[
            xspec(0), xspec(1), xspec(2),
            pl.BlockSpec((KS, HID, 3 * OUT_CH), lambda bi, t, n: (0, 0, 0)),
            pl.BlockSpec((1, 3 * OUT_CH), lambda bi, t, n: (0, 0)),
            pl.BlockSpec((NT, 4), lambda bi, t, n: (n, 0)),
        ],
        out_specs=pl.BlockSpec((1, 1, NT, OUT_CH), lambda bi, t, n: (bi, t, n, 0)),
        out_shape=jax.ShapeDtypeStruct((B, To, N, OUT_CH), jnp.float32),
        interpret=INTERPRET,
    )(zr, zr, zr, wcat, bcat, bn.T)


def _tconv2_mean_call(z, wcat, bcat, bn):
    S_in = z.shape[0]
    T = S_in // B
    To = T - 2
    zr = z.reshape(B, T, N, HID)

    def body(x0, x1, x2, w, b, bnr, out):
        t = pl.program_id(2)
        y = (jnp.dot(x0[0, 0], w[0], preferred_element_type=jnp.float32)
             + jnp.dot(x1[0, 0], w[1], preferred_element_type=jnp.float32)
             + jnp.dot(x2[0, 0], w[2], preferred_element_type=jnp.float32)
             + b[0])
        g = jax.nn.relu(y[:, :OUT_CH] * jax.nn.sigmoid(y[:, OUT_CH:2 * OUT_CH])
                        + y[:, 2 * OUT_CH:])
        gam, bet, m, v = bnr[:, 0:1], bnr[:, 1:2], bnr[:, 2:3], bnr[:, 3:4]
        scale = gam * lax.rsqrt(v + 1e-5)
        shift = bet - m * scale
        val = jax.nn.relu(g * scale + shift) * (1.0 / To)

        @pl.when(t == 0)
        def _():
            out[0] = jnp.zeros_like(out[0])
        out[0] += val

    xspec = lambda k: pl.BlockSpec((1, 1, NT, HID), lambda bi, n, t: (bi, t + k, n, 0))
    return pl.pallas_call(
        body,
        grid=(B, NG, To),
        in_specs=[
            xspec(0), xspec(1), xspec(2),
            pl.BlockSpec((KS, HID, 3 * OUT_CH), lambda bi, n, t: (0, 0, 0)),
            pl.BlockSpec((1, 3 * OUT_CH), lambda bi, n, t: (0, 0)),
            pl.BlockSpec((NT, 4), lambda bi, n, t: (n, 0)),
        ],
        out_specs=pl.BlockSpec((1, NT, OUT_CH), lambda bi, n, t: (bi, n, 0)),
        out_shape=jax.ShapeDtypeStruct((B, N, OUT_CH), jnp.float32),
        interpret=INTERPRET,
    )(zr, zr, zr, wcat, bcat, bn.T)


def _classifier_call(h, w1, b1, w2, b2):
    K = h.shape[1]
    KC = 16000
    KG = K // KC

    def body(hr, w1r, b1r, w2r, b2r, out, acc):
        k = pl.program_id(0)

        @pl.when(k == 0)
        def _():
            acc[...] = jnp.zeros_like(acc[...])
        acc[...] += lax.dot_general(hr[...], w1r[...],
                                    (((1,), (1,)), ((), ())),
                                    preferred_element_type=jnp.float32)

        @pl.when(k == KG - 1)
        def _():
            zz = jax.nn.relu(acc[...] + b1r[...])
            out[...] = lax.dot_general(zz, w2r[...],
                                       (((1,), (1,)), ((), ())),
                                       preferred_element_type=jnp.float32) + b2r[...]

    return pl.pallas_call(
        body,
        grid=(KG,),
        in_specs=[
            pl.BlockSpec((B, KC), lambda k: (0, k)),
            pl.BlockSpec((256, KC), lambda k: (0, k)),
            pl.BlockSpec((1, 256), lambda k: (0, 0)),
            pl.BlockSpec((NCLS, 256), lambda k: (0, 0)),
            pl.BlockSpec((1, NCLS), lambda k: (0, 0)),
        ],
        out_specs=pl.BlockSpec((B, NCLS), lambda k: (0, 0)),
        out_shape=jax.ShapeDtypeStruct((B, NCLS), jnp.float32),
        scratch_shapes=[pltpu.VMEM((B, 256), jnp.float32)],
        interpret=INTERPRET,
    )(h, w1, b1.reshape(1, 256), w2, b2.reshape(1, NCLS))


NC = 2
NS = 16
E = 160000
GB = 256
TSTEPS = 40
EPT = TSTEPS * GB
NPAD = NS * 632
ZSTR = 632
ZSUB = 320
WLAST = N - ZSTR * (NS - 1)

_SC_MESH = plsc.VectorSubcoreMesh(core_axis_name="c", subcore_axis_name="s")


def _prop_sc(tab, rowp, colp, zeros):
    S = tab.shape[0]
    SH = S // NC

    @functools.partial(
        pl.kernel,
        out_type=jax.ShapeDtypeStruct((S, N, HID), jnp.float32),
        mesh=_SC_MESH,
        compiler_params=pltpu.CompilerParams(use_tc_tiling_on_sc=False),
        scratch_types=[
            pltpu.VMEM((TSTEPS, GB), jnp.int32),
            pltpu.VMEM((TSTEPS, GB), jnp.int32),
            pltpu.VMEM((ZSUB, HID), jnp.float32),
            pltpu.VMEM((3, GB, HID), jnp.float32),
            pltpu.VMEM_SHARED((NPAD, HID), jnp.float32),
            pltpu.SemaphoreType.DMA((3,)),
            pltpu.SemaphoreType.DMA((3,)),
        ],
    )
    def run(tab_hbm, rowp_hbm, colp_hbm, zeros_hbm, out_hbm,
            row_v, col_v, zeros_v, rows_v, y_sh, gsem, ssem):
        core = lax.axis_index("c")
        sid = lax.axis_index("s")
        pltpu.sync_copy(rowp_hbm.at[pl.ds(sid * TSTEPS, TSTEPS)], row_v)
        pltpu.sync_copy(colp_hbm.at[pl.ds(sid * TSTEPS, TSTEPS)], col_v)
        pltpu.sync_copy(zeros_hbm, zeros_v)

        def chunk(ci, carry):
            s = ci * NC + core
            pltpu.sync_copy(zeros_v, y_sh.at[pl.ds(sid * ZSTR, ZSUB)])
            pltpu.sync_copy(zeros_v.at[pl.ds(0, ZSTR - ZSUB)],
                            y_sh.at[pl.ds(sid * ZSTR + ZSUB, ZSTR - ZSUB)])
            for b in range(2):
                pltpu.async_copy(tab_hbm.at[s].at[row_v.at[b]],
                                 rows_v.at[b], gsem.at[b])
            plsc.subcore_barrier()

            def step(j, c2):
                b = lax.rem(j, 3)
                pltpu.make_async_copy(tab_hbm.at[s].at[row_v.at[j]],
                                      rows_v.at[b], gsem.at[b]).wait()
                pltpu.async_copy(rows_v.at[b], y_sh.at[col_v.at[j]],
                                 ssem.at[b], add=True)
                pb = lax.rem(j + 2, 3)

                @pl.when((j >= 1) & (j + 2 < TSTEPS))
                def _():
                    pltpu.make_async_copy(rows_v.at[pb],
                                          y_sh.at[col_v.at[j - 1]],
                                          ssem.at[pb]).wait()

                @pl.when(j + 2 < TSTEPS)
                def _():
                    pltpu.async_copy(tab_hbm.at[s].at[row_v.at[j + 2]],
                                     rows_v.at[pb], gsem.at[pb])
                return c2
            lax.fori_loop(0, TSTEPS, step, 0, unroll=False)
            for jj in range(TSTEPS - 3, TSTEPS):
                pltpu.make_async_copy(rows_v.at[jj % 3], y_sh.at[col_v.at[jj]],
                                      ssem.at[jj % 3]).wait()
            plsc.subcore_barrier()

            @pl.when(sid < NS - 1)
            def _():
                pltpu.sync_copy(y_sh.at[pl.ds(sid * ZSTR, ZSTR)],
                                out_hbm.at[s].at[pl.ds(sid * ZSTR, ZSTR)])

            @pl.when(sid == NS - 1)
            def _():
                pltpu.sync_copy(y_sh.at[pl.ds((NS - 1) * ZSTR, WLAST)],
                                out_hbm.at[s].at[pl.ds((NS - 1) * ZSTR, WLAST)])
            plsc.subcore_barrier()
            return carry

        lax.fori_loop(0, SH, chunk, 0, unroll=False)

    return run(tab, rowp, colp, zeros)


def _wcat_t(tw):
    KSs, cout, cin = tw.shape[4], tw.shape[1], tw.shape[2]
    return jnp.transpose(tw[:, :, :, 0, :], (3, 2, 0, 1)).reshape(KSs, cin, 3 * cout)


def _stconv_block(x, rowp, colp, zeros, deg, t1w, t1b, chw, chb, t2w, t2b, bn, mean):
    xg, xs = _tconv1_call(x, _wcat_t(t1w), t1b.reshape(1, -1), deg)
    g1 = _prop_sc(xs.reshape(-1, N, HID), rowp, colp, zeros)
    tab2 = _scale_call(g1, deg)
    g2 = _prop_sc(tab2.reshape(-1, N, HID), rowp, colp, zeros)
    z = _cheb_combine_call(xg, g1, g2, deg, chw, chb)
    if mean:
        return _tconv2_mean_call(z, _wcat_t(t2w), t2b.reshape(1, -1), bn)
    return _tconv2_call(z, _wcat_t(t2w), t2b.reshape(1, -1), bn)


def kernel(x, edge_index, s1_t1w, s1_t1b, s1_chw, s1_chb, s1_t2w, s1_t2b, s1_bn,
           s2_t1w, s2_t1b, s2_chw, s2_chb, s2_t2w, s2_t2b, s2_bn,
           cls_w1, cls_b1, cls_w2, cls_b2):
    row = edge_index[0]
    col = edge_index[1]

    npad = NS * TSTEPS * GB - E
    rowp = jnp.concatenate([row, jnp.zeros((npad,), jnp.int32)]).reshape(NS * TSTEPS, GB)
    colp = jnp.concatenate([col, jnp.full((npad,), N, jnp.int32)]).reshape(NS * TSTEPS, GB)
    rowd = jnp.concatenate([row, jnp.full((npad,), N, jnp.int32)]).reshape(NS * TSTEPS, GB)
    zeros = jnp.zeros((ZSUB, HID), jnp.float32)

    deg = _prop_sc(jnp.ones((NC, N, HID), jnp.float32), rowp, rowd, zeros)[0, :, 0:1]

    h1 = _stconv_block(x, rowp, colp, zeros, deg, s1_t1w, s1_t1b, s1_chw, s1_chb,
                       s1_t2w, s1_t2b, s1_bn, mean=False)
    hm = _stconv_block(h1, rowp, colp, zeros, deg, s2_t1w, s2_t1b, s2_chw, s2_chb,
                       s2_t2w, s2_t2b, s2_bn, mean=True)
    return _classifier_call(hm.reshape(B, N * OUT_CH), cls_w1, cls_b1, cls_w2, cls_b2)


# speedup vs baseline: 1.4535x; 1.4535x over previous
import functools

import jax
import jax.numpy as jnp
from jax import lax
from jax.experimental import pallas as pl
from jax.experimental.pallas import tpu as pltpu
from jax.experimental.pallas import tpu_sc as plsc

B = 2
TSEQ = 16
N = 10000
IN_CH = 3
HID = 64
OUT_CH = 16
KS = 3
NCLS = 10

NT = 2000
NG = N // NT

INTERPRET = False


def _tconv1_call(x, wcat, bcat, deg):
    Bb, T, _, cin = x.shape
    To = T - 2
    S = Bb * To

    def body(x0, x1, x2, w, b, dg, out, outs):
        y = (jnp.dot(x0[0, 0], w[0], preferred_element_type=jnp.float32)
             + jnp.dot(x1[0, 0], w[1], preferred_element_type=jnp.float32)
             + jnp.dot(x2[0, 0], w[2], preferred_element_type=jnp.float32)
             + b[0])
        g = jax.nn.relu(y[:, :HID] * jax.nn.sigmoid(y[:, HID:2 * HID])
                        + y[:, 2 * HID:])
        dis = jnp.where(dg[...] > 0, lax.rsqrt(dg[...]), 0.0)
        out[0] = g
        outs[0] = (g * dis).astype(jnp.bfloat16)

    xspec = lambda k: pl.BlockSpec((1, 1, NT, cin), lambda bi, t, n: (bi, t + k, n, 0))
    return pl.pallas_call(
        body,
        grid=(Bb, To, NG),
        in_specs=[
            xspec(0), xspec(1), xspec(2),
            pl.BlockSpec((KS, cin, 3 * HID), lambda bi, t, n: (0, 0, 0)),
            pl.BlockSpec((1, 3 * HID), lambda bi, t, n: (0, 0)),
            pl.BlockSpec((NT, 1), lambda bi, t, n: (n, 0)),
        ],
        out_specs=[
            pl.BlockSpec((1, NT, HID), lambda bi, t, n: (bi * To + t, n, 0)),
            pl.BlockSpec((1, NT, HID), lambda bi, t, n: (bi * To + t, n, 0)),
        ],
        out_shape=[
            jax.ShapeDtypeStruct((S, N, HID), jnp.float32),
            jax.ShapeDtypeStruct((S, N, HID), jnp.bfloat16),
        ],
        interpret=INTERPRET,
    )(x, x, x, wcat, bcat, deg)


def _cheb_combine_call(xc, g1, g2, deg, chw, chb):
    S = xc.shape[0]

    def body(xr, g1r, g2r, dg, w, b, out):
        dis = jnp.where(dg[...] > 0, lax.rsqrt(dg[...]), 0.0)
        x = xr[0]
        a = g1r[0].astype(jnp.float32) * (-dis)
        c = g2r[0].astype(jnp.float32) * (2.0 * dis)
        z = (jnp.dot(x, w[0] - w[2], preferred_element_type=jnp.float32)
             + jnp.dot(a, w[1], preferred_element_type=jnp.float32)
             + jnp.dot(c, w[2], preferred_element_type=jnp.float32)
             + b[0])
        out[0] = jax.nn.relu(z)

    fspec = pl.BlockSpec((1, NT, HID), lambda s, n: (s, n, 0))
    return pl.pallas_call(
        body,
        grid=(S, NG),
        in_specs=[
            fspec, fspec, fspec,
            pl.BlockSpec((NT, 1), lambda s, n: (n, 0)),
            pl.BlockSpec((3, HID, HID), lambda s, n: (0, 0, 0)),
            pl.BlockSpec((1, HID), lambda s, n: (0, 0)),
        ],
        out_specs=fspec,
        out_shape=jax.ShapeDtypeStruct((S, N, HID), jnp.float32),
        interpret=INTERPRET,
    )(xc, g1, g2, deg, chw, chb.reshape(1, HID))


def _scale_call(g1, deg):
    S = g1.shape[0]

    def body(g1r, dg, out):
        inv = jnp.where(dg[...] > 0, 1.0 / dg[...], 0.0)
        out[0] = (g1r[0].astype(jnp.float32) * inv).astype(jnp.bfloat16)

    fspec = pl.BlockSpec((1, NT, HID), lambda s, n: (s, n, 0))
    return pl.pallas_call(
        body,
        grid=(S, NG),
        in_specs=[fspec, pl.BlockSpec((NT, 1), lambda s, n: (n, 0))],
        out_specs=fspec,
        out_shape=jax.ShapeDtypeStruct((S, N, HID), jnp.bfloat16),
        interpret=INTERPRET,
    )(g1, deg)


def _tconv2_call(z, wcat, bcat, bn):
    S_in = z.shape[0]
    T = S_in // B
    To = T - 2
    zr = z.reshape(B, T, N, HID)

    def body(x0, x1, x2, w, b, bnr, out):
        y = (jnp.dot(x0[0, 0], w[0], preferred_element_type=jnp.float32)
             + jnp.dot(x1[0, 0], w[1], preferred_element_type=jnp.float32)
             + jnp.dot(x2[0, 0], w[2], preferred_element_type=jnp.float32)
             + b[0])
        g = jax.nn.relu(y[:, :OUT_CH] * jax.nn.sigmoid(y[:, OUT_CH:2 * OUT_CH])
                        + y[:, 2 * OUT_CH:])
        gam, bet, m, v = bnr[:, 0:1], bnr[:, 1:2], bnr[:, 2:3], bnr[:, 3:4]
        scale = gam * lax.rsqrt(v + 1e-5)
        shift = bet - m * scale
        out[0, 0] = jax.nn.relu(g * scale + shift)

    xspec = lambda k: pl.BlockSpec((1, 1, NT, HID), lambda bi, t, n: (bi, t + k, n, 0))
    return pl.pallas_call(
        body,
        grid=(B, To, NG),
        in_specs=[
            xspec(0), xspec(1), xspec(2),
            pl.BlockSpec((KS, HID, 3 * OUT_CH), lambda bi, t, n: (0, 0, 0)),
            pl.BlockSpec((1, 3 * OUT_CH), lambda bi, t, n: (0, 0)),
            pl.BlockSpec((NT, 4), lambda bi, t, n: (n, 0)),
        ],
        out_specs=pl.BlockSpec((1, 1, NT, OUT_CH), lambda bi, t, n: (bi, t, n, 0)),
        out_shape=jax.ShapeDtypeStruct((B, To, N, OUT_CH), jnp.float32),
        interpret=INTERPRET,
    )(zr, zr, zr, wcat, bcat, bn.T)


def _tconv2_mean_call(z, wcat, bcat, bn):
    S_in = z.shape[0]
    T = S_in // B
    To = T - 2
    zr = z.reshape(B, T, N, HID)

    def body(x0, x1, x2, w, b, bnr, out):
        t = pl.program_id(2)
        y = (jnp.dot(x0[0, 0], w[0], preferred_element_type=jnp.float32)
             + jnp.dot(x1[0, 0], w[1], preferred_element_type=jnp.float32)
             + jnp.dot(x2[0, 0], w[2], preferred_element_type=jnp.float32)
             + b[0])
        g = jax.nn.relu(y[:, :OUT_CH] * jax.nn.sigmoid(y[:, OUT_CH:2 * OUT_CH])
                        + y[:, 2 * OUT_CH:])
        gam, bet, m, v = bnr[:, 0:1], bnr[:, 1:2], bnr[:, 2:3], bnr[:, 3:4]
        scale = gam * lax.rsqrt(v + 1e-5)
        shift = bet - m * scale
        val = jax.nn.relu(g * scale + shift) * (1.0 / To)

        @pl.when(t == 0)
        def _():
            out[0] = jnp.zeros_like(out[0])
        out[0] += val

    xspec = lambda k: pl.BlockSpec((1, 1, NT, HID), lambda bi, n, t: (bi, t + k, n, 0))
    return pl.pallas_call(
        body,
        grid=(B, NG, To),
        in_specs=[
            xspec(0), xspec(1), xspec(2),
            pl.BlockSpec((KS, HID, 3 * OUT_CH), lambda bi, n, t: (0, 0, 0)),
            pl.BlockSpec((1, 3 * OUT_CH), lambda bi, n, t: (0, 0)),
            pl.BlockSpec((NT, 4), lambda bi, n, t: (n, 0)),
        ],
        out_specs=pl.BlockSpec((1, NT, OUT_CH), lambda bi, n, t: (bi, n, 0)),
        out_shape=jax.ShapeDtypeStruct((B, N, OUT_CH), jnp.float32),
        interpret=INTERPRET,
    )(zr, zr, zr, wcat, bcat, bn.T)


def _classifier_call(h, w1, b1, w2, b2):
    K = h.shape[1]
    KC = 16000
    KG = K // KC

    def body(hr, w1r, b1r, w2r, b2r, out, acc):
        k = pl.program_id(0)

        @pl.when(k == 0)
        def _():
            acc[...] = jnp.zeros_like(acc[...])
        acc[...] += lax.dot_general(hr[...], w1r[...],
                                    (((1,), (1,)), ((), ())),
                                    preferred_element_type=jnp.float32)

        @pl.when(k == KG - 1)
        def _():
            zz = jax.nn.relu(acc[...] + b1r[...])
            out[...] = lax.dot_general(zz, w2r[...],
                                       (((1,), (1,)), ((), ())),
                                       preferred_element_type=jnp.float32) + b2r[...]

    return pl.pallas_call(
        body,
        grid=(KG,),
        in_specs=[
            pl.BlockSpec((B, KC), lambda k: (0, k)),
            pl.BlockSpec((256, KC), lambda k: (0, k)),
            pl.BlockSpec((1, 256), lambda k: (0, 0)),
            pl.BlockSpec((NCLS, 256), lambda k: (0, 0)),
            pl.BlockSpec((1, NCLS), lambda k: (0, 0)),
        ],
        out_specs=pl.BlockSpec((B, NCLS), lambda k: (0, 0)),
        out_shape=jax.ShapeDtypeStruct((B, NCLS), jnp.float32),
        scratch_shapes=[pltpu.VMEM((B, 256), jnp.float32)],
        interpret=INTERPRET,
    )(h, w1, b1.reshape(1, 256), w2, b2.reshape(1, NCLS))


NC = 2
NS = 16
E = 160000
GB = 256
TSTEPS = 40
EPT = TSTEPS * GB
NPAD = NS * 632
ZSTR = 632
ZSUB = 320
WLAST = N - ZSTR * (NS - 1)

_SC_MESH = plsc.VectorSubcoreMesh(core_axis_name="c", subcore_axis_name="s")


def _prop_sc(tab, rowp, colp, zeros):
    S = tab.shape[0]
    SH = S // NC

    @functools.partial(
        pl.kernel,
        out_type=jax.ShapeDtypeStruct((S, N, HID), jnp.bfloat16),
        mesh=_SC_MESH,
        compiler_params=pltpu.CompilerParams(use_tc_tiling_on_sc=False),
        scratch_types=[
            pltpu.VMEM((TSTEPS, GB), jnp.int32),
            pltpu.VMEM((TSTEPS, GB), jnp.int32),
            pltpu.VMEM((ZSUB, HID), jnp.bfloat16),
            pltpu.VMEM((3, GB, HID), jnp.bfloat16),
            pltpu.VMEM_SHARED((NPAD, HID), jnp.bfloat16),
            pltpu.SemaphoreType.DMA((3,)),
            pltpu.SemaphoreType.DMA((3,)),
        ],
    )
    def run(tab_hbm, rowp_hbm, colp_hbm, zeros_hbm, out_hbm,
            row_v, col_v, zeros_v, rows_v, y_sh, gsem, ssem):
        core = lax.axis_index("c")
        sid = lax.axis_index("s")
        pltpu.sync_copy(rowp_hbm.at[pl.ds(sid * TSTEPS, TSTEPS)], row_v)
        pltpu.sync_copy(colp_hbm.at[pl.ds(sid * TSTEPS, TSTEPS)], col_v)
        pltpu.sync_copy(zeros_hbm, zeros_v)

        def chunk(ci, carry):
            s = ci * NC + core
            pltpu.sync_copy(zeros_v, y_sh.at[pl.ds(sid * ZSTR, ZSUB)])
            pltpu.sync_copy(zeros_v.at[pl.ds(0, ZSTR - ZSUB)],
                            y_sh.at[pl.ds(sid * ZSTR + ZSUB, ZSTR - ZSUB)])
            for b in range(2):
                pltpu.async_copy(tab_hbm.at[s].at[row_v.at[b]],
                                 rows_v.at[b], gsem.at[b])
            plsc.subcore_barrier()

            def step(j, c2):
                b = lax.rem(j, 3)
                pltpu.make_async_copy(tab_hbm.at[s].at[row_v.at[j]],
                                      rows_v.at[b], gsem.at[b]).wait()
                pltpu.async_copy(rows_v.at[b], y_sh.at[col_v.at[j]],
                                 ssem.at[b], add=True)
                pb = lax.rem(j + 2, 3)

                @pl.when((j >= 1) & (j + 2 < TSTEPS))
                def _():
                    pltpu.make_async_copy(rows_v.at[pb],
                                          y_sh.at[col_v.at[j - 1]],
                                          ssem.at[pb]).wait()

                @pl.when(j + 2 < TSTEPS)
                def _():
                    pltpu.async_copy(tab_hbm.at[s].at[row_v.at[j + 2]],
                                     rows_v.at[pb], gsem.at[pb])
                return c2
            lax.fori_loop(0, TSTEPS, step, 0, unroll=False)
            for jj in range(TSTEPS - 3, TSTEPS):
                pltpu.make_async_copy(rows_v.at[jj % 3], y_sh.at[col_v.at[jj]],
                                      ssem.at[jj % 3]).wait()
            plsc.subcore_barrier()

            @pl.when(sid < NS - 1)
            def _():
                pltpu.sync_copy(y_sh.at[pl.ds(sid * ZSTR, ZSTR)],
                                out_hbm.at[s].at[pl.ds(sid * ZSTR, ZSTR)])

            @pl.when(sid == NS - 1)
            def _():
                pltpu.sync_copy(y_sh.at[pl.ds((NS - 1) * ZSTR, WLAST)],
                                out_hbm.at[s].at[pl.ds((NS - 1) * ZSTR, WLAST)])
            plsc.subcore_barrier()
            return carry

        lax.fori_loop(0, SH, chunk, 0, unroll=False)

    return run(tab, rowp, colp, zeros)


def _wcat_t(tw):
    KSs, cout, cin = tw.shape[4], tw.shape[1], tw.shape[2]
    return jnp.transpose(tw[:, :, :, 0, :], (3, 2, 0, 1)).reshape(KSs, cin, 3 * cout)


def _stconv_block(x, rowp, colp, zeros, deg, t1w, t1b, chw, chb, t2w, t2b, bn, mean):
    xg, xs = _tconv1_call(x, _wcat_t(t1w), t1b.reshape(1, -1), deg)
    g1 = _prop_sc(xs.reshape(-1, N, HID), rowp, colp, zeros)
    tab2 = _scale_call(g1, deg)
    g2 = _prop_sc(tab2.reshape(-1, N, HID), rowp, colp, zeros)
    z = _cheb_combine_call(xg, g1, g2, deg, chw, chb)
    if mean:
        return _tconv2_mean_call(z, _wcat_t(t2w), t2b.reshape(1, -1), bn)
    return _tconv2_call(z, _wcat_t(t2w), t2b.reshape(1, -1), bn)


def kernel(x, edge_index, s1_t1w, s1_t1b, s1_chw, s1_chb, s1_t2w, s1_t2b, s1_bn,
           s2_t1w, s2_t1b, s2_chw, s2_chb, s2_t2w, s2_t2b, s2_bn,
           cls_w1, cls_b1, cls_w2, cls_b2):
    row = edge_index[0]
    col = edge_index[1]

    npad = NS * TSTEPS * GB - E
    rowp = jnp.concatenate([row, jnp.zeros((npad,), jnp.int32)]).reshape(NS * TSTEPS, GB)
    colp = jnp.concatenate([col, jnp.full((npad,), N, jnp.int32)]).reshape(NS * TSTEPS, GB)
    rowd = jnp.concatenate([row, jnp.full((npad,), N, jnp.int32)]).reshape(NS * TSTEPS, GB)
    zeros = jnp.zeros((ZSUB, HID), jnp.bfloat16)

    deg = _prop_sc(jnp.ones((NC, N, HID), jnp.bfloat16), rowp, rowd,
                   zeros)[0, :, 0:1].astype(jnp.float32)

    h1 = _stconv_block(x, rowp, colp, zeros, deg, s1_t1w, s1_t1b, s1_chw, s1_chb,
                       s1_t2w, s1_t2b, s1_bn, mean=False)
    hm = _stconv_block(h1, rowp, colp, zeros, deg, s2_t1w, s2_t1b, s2_chw, s2_chb,
                       s2_t2w, s2_t2b, s2_bn, mean=True)
    return _classifier_call(hm.reshape(B, N * OUT_CH), cls_w1, cls_b1, cls_w2, cls_b2)


# speedup vs baseline: 1.5147x; 1.0421x over previous
import functools

import jax
import jax.numpy as jnp
from jax import lax
from jax.experimental import pallas as pl
from jax.experimental.pallas import tpu as pltpu
from jax.experimental.pallas import tpu_sc as plsc

B = 2
TSEQ = 16
N = 10000
IN_CH = 3
HID = 64
OUT_CH = 16
KS = 3
NCLS = 10

NT = 5000
NG = N // NT

INTERPRET = False


def _tconv1_call(x, wcat, bcat, deg):
    Bb, T, _, cin = x.shape
    To = T - 2
    S = Bb * To

    def body(x0, x1, x2, w, b, dg, out, outs):
        y = (jnp.dot(x0[0, 0], w[0], preferred_element_type=jnp.float32)
             + jnp.dot(x1[0, 0], w[1], preferred_element_type=jnp.float32)
             + jnp.dot(x2[0, 0], w[2], preferred_element_type=jnp.float32)
             + b[0])
        g = jax.nn.relu(y[:, :HID] * jax.nn.sigmoid(y[:, HID:2 * HID])
                        + y[:, 2 * HID:])
        dis = jnp.where(dg[...] > 0, lax.rsqrt(dg[...]), 0.0)
        out[0] = g
        outs[0] = (g * dis).astype(jnp.bfloat16)

    xspec = lambda k: pl.BlockSpec((1, 1, NT, cin), lambda bi, t, n: (bi, t + k, n, 0))
    return pl.pallas_call(
        body,
        grid=(Bb, To, NG),
        in_specs=[
            xspec(0), xspec(1), xspec(2),
            pl.BlockSpec((KS, cin, 3 * HID), lambda bi, t, n: (0, 0, 0)),
            pl.BlockSpec((1, 3 * HID), lambda bi, t, n: (0, 0)),
            pl.BlockSpec((NT, 1), lambda bi, t, n: (n, 0)),
        ],
        out_specs=[
            pl.BlockSpec((1, NT, HID), lambda bi, t, n: (bi * To + t, n, 0)),
            pl.BlockSpec((1, NT, HID), lambda bi, t, n: (bi * To + t, n, 0)),
        ],
        out_shape=[
            jax.ShapeDtypeStruct((S, N, HID), jnp.float32),
            jax.ShapeDtypeStruct((S, N, HID), jnp.bfloat16),
        ],
        interpret=INTERPRET,
    )(x, x, x, wcat, bcat, deg)


def _cheb_combine_call(xc, g1, g2, deg, chw, chb):
    S = xc.shape[0]

    def body(xr, g1r, g2r, dg, w, b, out):
        dis = jnp.where(dg[...] > 0, lax.rsqrt(dg[...]), 0.0)
        x = xr[0]
        a = g1r[0].astype(jnp.float32) * (-dis)
        c = g2r[0].astype(jnp.float32) * (2.0 * dis)
        z = (jnp.dot(x, w[0] - w[2], preferred_element_type=jnp.float32)
             + jnp.dot(a, w[1], preferred_element_type=jnp.float32)
             + jnp.dot(c, w[2], preferred_element_type=jnp.float32)
             + b[0])
        out[0] = jax.nn.relu(z)

    fspec = pl.BlockSpec((1, NT, HID), lambda s, n: (s, n, 0))
    return pl.pallas_call(
        body,
        grid=(S, NG),
        in_specs=[
            fspec, fspec, fspec,
            pl.BlockSpec((NT, 1), lambda s, n: (n, 0)),
            pl.BlockSpec((3, HID, HID), lambda s, n: (0, 0, 0)),
            pl.BlockSpec((1, HID), lambda s, n: (0, 0)),
        ],
        out_specs=fspec,
        out_shape=jax.ShapeDtypeStruct((S, N, HID), jnp.float32),
        interpret=INTERPRET,
    )(xc, g1, g2, deg, chw, chb.reshape(1, HID))


def _scale_call(g1, deg):
    S = g1.shape[0]

    def body(g1r, dg, out):
        inv = jnp.where(dg[...] > 0, 1.0 / dg[...], 0.0)
        out[0] = (g1r[0].astype(jnp.float32) * inv).astype(jnp.bfloat16)

    fspec = pl.BlockSpec((1, NT, HID), lambda s, n: (s, n, 0))
    return pl.pallas_call(
        body,
        grid=(S, NG),
        in_specs=[fspec, pl.BlockSpec((NT, 1), lambda s, n: (n, 0))],
        out_specs=fspec,
        out_shape=jax.ShapeDtypeStruct((S, N, HID), jnp.bfloat16),
        interpret=INTERPRET,
    )(g1, deg)


def _tconv2_call(z, wcat, bcat, bn):
    S_in = z.shape[0]
    T = S_in // B
    To = T - 2
    zr = z.reshape(B, T, N, HID)

    def body(x0, x1, x2, w, b, bnr, out):
        y = (jnp.dot(x0[0, 0], w[0], preferred_element_type=jnp.float32)
             + jnp.dot(x1[0, 0], w[1], preferred_element_type=jnp.float32)
             + jnp.dot(x2[0, 0], w[2], preferred_element_type=jnp.float32)
             + b[0])
        g = jax.nn.relu(y[:, :OUT_CH] * jax.nn.sigmoid(y[:, OUT_CH:2 * OUT_CH])
                        + y[:, 2 * OUT_CH:])
        gam, bet, m, v = bnr[:, 0:1], bnr[:, 1:2], bnr[:, 2:3], bnr[:, 3:4]
        scale = gam * lax.rsqrt(v + 1e-5)
        shift = bet - m * scale
        out[0, 0] = jax.nn.relu(g * scale + shift)

    xspec = lambda k: pl.BlockSpec((1, 1, NT, HID), lambda bi, t, n: (bi, t + k, n, 0))
    return pl.pallas_call(
        body,
        grid=(B, To, NG),
        in_specs=[
            xspec(0), xspec(1), xspec(2),
            pl.BlockSpec((KS, HID, 3 * OUT_CH), lambda bi, t, n: (0, 0, 0)),
            pl.BlockSpec((1, 3 * OUT_CH), lambda bi, t, n: (0, 0)),
            pl.BlockSpec((NT, 4), lambda bi, t, n: (n, 0)),
        ],
        out_specs=pl.BlockSpec((1, 1, NT, OUT_CH), lambda bi, t, n: (bi, t, n, 0)),
        out_shape=jax.ShapeDtypeStruct((B, To, N, OUT_CH), jnp.float32),
        interpret=INTERPRET,
    )(zr, zr, zr, wcat, bcat, bn.T)


def _tconv2_mean_call(z, wcat, bcat, bn):
    S_in = z.shape[0]
    T = S_in // B
    To = T - 2
    zr = z.reshape(B, T, N, HID)

    def body(x0, x1, x2, w, b, bnr, out):
        t = pl.program_id(2)
        y = (jnp.dot(x0[0, 0], w[0], preferred_element_type=jnp.float32)
             + jnp.dot(x1[0, 0], w[1], preferred_element_type=jnp.float32)
             + jnp.dot(x2[0, 0], w[2], preferred_element_type=jnp.float32)
             + b[0])
        g = jax.nn.relu(y[:, :OUT_CH] * jax.nn.sigmoid(y[:, OUT_CH:2 * OUT_CH])
                        + y[:, 2 * OUT_CH:])
        gam, bet, m, v = bnr[:, 0:1], bnr[:, 1:2], bnr[:, 2:3], bnr[:, 3:4]
        scale = gam * lax.rsqrt(v + 1e-5)
        shift = bet - m * scale
        val = jax.nn.relu(g * scale + shift) * (1.0 / To)

        @pl.when(t == 0)
        def _():
            out[0] = jnp.zeros_like(out[0])
        out[0] += val

    xspec = lambda k: pl.BlockSpec((1, 1, NT, HID), lambda bi, n, t: (bi, t + k, n, 0))
    return pl.pallas_call(
        body,
        grid=(B, NG, To),
        in_specs=[
            xspec(0), xspec(1), xspec(2),
            pl.BlockSpec((KS, HID, 3 * OUT_CH), lambda bi, n, t: (0, 0, 0)),
            pl.BlockSpec((1, 3 * OUT_CH), lambda bi, n, t: (0, 0)),
            pl.BlockSpec((NT, 4), lambda bi, n, t: (n, 0)),
        ],
        out_specs=pl.BlockSpec((1, NT, OUT_CH), lambda bi, n, t: (bi, n, 0)),
        out_shape=jax.ShapeDtypeStruct((B, N, OUT_CH), jnp.float32),
        interpret=INTERPRET,
    )(zr, zr, zr, wcat, bcat, bn.T)


def _classifier_call(h, w1, b1, w2, b2):
    K = h.shape[1]
    KC = 16000
    KG = K // KC

    def body(hr, w1r, b1r, w2r, b2r, out, acc):
        k = pl.program_id(0)

        @pl.when(k == 0)
        def _():
            acc[...] = jnp.zeros_like(acc[...])
        acc[...] += lax.dot_general(hr[...], w1r[...],
                                    (((1,), (1,)), ((), ())),
                                    preferred_element_type=jnp.float32)

        @pl.when(k == KG - 1)
        def _():
            zz = jax.nn.relu(acc[...] + b1r[...])
            out[...] = lax.dot_general(zz, w2r[...],
                                       (((1,), (1,)), ((), ())),
                                       preferred_element_type=jnp.float32) + b2r[...]

    return pl.pallas_call(
        body,
        grid=(KG,),
        in_specs=[
            pl.BlockSpec((B, KC), lambda k: (0, k)),
            pl.BlockSpec((256, KC), lambda k: (0, k)),
            pl.BlockSpec((1, 256), lambda k: (0, 0)),
            pl.BlockSpec((NCLS, 256), lambda k: (0, 0)),
            pl.BlockSpec((1, NCLS), lambda k: (0, 0)),
        ],
        out_specs=pl.BlockSpec((B, NCLS), lambda k: (0, 0)),
        out_shape=jax.ShapeDtypeStruct((B, NCLS), jnp.float32),
        scratch_shapes=[pltpu.VMEM((B, 256), jnp.float32)],
        interpret=INTERPRET,
    )(h, w1, b1.reshape(1, 256), w2, b2.reshape(1, NCLS))


NC = 2
NS = 16
E = 160000
GB = 256
TSTEPS = 40
EPT = TSTEPS * GB
NPAD = NS * 632
ZSTR = 632
ZSUB = 320
WLAST = N - ZSTR * (NS - 1)

_SC_MESH = plsc.VectorSubcoreMesh(core_axis_name="c", subcore_axis_name="s")


def _prop_sc(tab, rowp, colp, zeros):
    S = tab.shape[0]
    SH = S // NC

    @functools.partial(
        pl.kernel,
        out_type=jax.ShapeDtypeStruct((S, N, HID), jnp.bfloat16),
        mesh=_SC_MESH,
        compiler_params=pltpu.CompilerParams(use_tc_tiling_on_sc=False),
        scratch_types=[
            pltpu.VMEM((TSTEPS, GB), jnp.int32),
            pltpu.VMEM((TSTEPS, GB), jnp.int32),
            pltpu.VMEM((ZSUB, HID), jnp.bfloat16),
            pltpu.VMEM((3, GB, HID), jnp.bfloat16),
            pltpu.VMEM_SHARED((NPAD, HID), jnp.bfloat16),
            pltpu.SemaphoreType.DMA((3,)),
            pltpu.SemaphoreType.DMA((3,)),
        ],
    )
    def run(tab_hbm, rowp_hbm, colp_hbm, zeros_hbm, out_hbm,
            row_v, col_v, zeros_v, rows_v, y_sh, gsem, ssem):
        core = lax.axis_index("c")
        sid = lax.axis_index("s")
        pltpu.sync_copy(rowp_hbm.at[pl.ds(sid * TSTEPS, TSTEPS)], row_v)
        pltpu.sync_copy(colp_hbm.at[pl.ds(sid * TSTEPS, TSTEPS)], col_v)
        pltpu.sync_copy(zeros_hbm, zeros_v)

        def chunk(ci, carry):
            s = ci * NC + core
            pltpu.sync_copy(zeros_v, y_sh.at[pl.ds(sid * ZSTR, ZSUB)])
            pltpu.sync_copy(zeros_v.at[pl.ds(0, ZSTR - ZSUB)],
                            y_sh.at[pl.ds(sid * ZSTR + ZSUB, ZSTR - ZSUB)])
            for b in range(2):
                pltpu.async_copy(tab_hbm.at[s].at[row_v.at[b]],
                                 rows_v.at[b], gsem.at[b])
            plsc.subcore_barrier()

            def step(j, c2):
                b = lax.rem(j, 3)
                pltpu.make_async_copy(tab_hbm.at[s].at[row_v.at[j]],
                                      rows_v.at[b], gsem.at[b]).wait()
                pltpu.async_copy(rows_v.at[b], y_sh.at[col_v.at[j]],
                                 ssem.at[b], add=True)
                pb = lax.rem(j + 2, 3)

                @pl.when((j >= 1) & (j + 2 < TSTEPS))
                def _():
                    pltpu.make_async_copy(rows_v.at[pb],
                                          y_sh.at[col_v.at[j - 1]],
                                          ssem.at[pb]).wait()

                @pl.when(j + 2 < TSTEPS)
                def _():
                    pltpu.async_copy(tab_hbm.at[s].at[row_v.at[j + 2]],
                                     rows_v.at[pb], gsem.at[pb])
                return c2
            lax.fori_loop(0, TSTEPS, step, 0, unroll=False)
            for jj in range(TSTEPS - 3, TSTEPS):
                pltpu.make_async_copy(rows_v.at[jj % 3], y_sh.at[col_v.at[jj]],
                                      ssem.at[jj % 3]).wait()
            plsc.subcore_barrier()

            @pl.when(sid < NS - 1)
            def _():
                pltpu.sync_copy(y_sh.at[pl.ds(sid * ZSTR, ZSTR)],
                                out_hbm.at[s].at[pl.ds(sid * ZSTR, ZSTR)])

            @pl.when(sid == NS - 1)
            def _():
                pltpu.sync_copy(y_sh.at[pl.ds((NS - 1) * ZSTR, WLAST)],
                                out_hbm.at[s].at[pl.ds((NS - 1) * ZSTR, WLAST)])
            plsc.subcore_barrier()
            return carry

        lax.fori_loop(0, SH, chunk, 0, unroll=False)

    return run(tab, rowp, colp, zeros)


def _wcat_t(tw):
    KSs, cout, cin = tw.shape[4], tw.shape[1], tw.shape[2]
    return jnp.transpose(tw[:, :, :, 0, :], (3, 2, 0, 1)).reshape(KSs, cin, 3 * cout)


def _stconv_block(x, rowp, colp, zeros, deg, t1w, t1b, chw, chb, t2w, t2b, bn, mean):
    xg, xs = _tconv1_call(x, _wcat_t(t1w), t1b.reshape(1, -1), deg)
    g1 = _prop_sc(xs.reshape(-1, N, HID), rowp, colp, zeros)
    tab2 = _scale_call(g1, deg)
    g2 = _prop_sc(tab2.reshape(-1, N, HID), rowp, colp, zeros)
    z = _cheb_combine_call(xg, g1, g2, deg, chw, chb)
    if mean:
        return _tconv2_mean_call(z, _wcat_t(t2w), t2b.reshape(1, -1), bn)
    return _tconv2_call(z, _wcat_t(t2w), t2b.reshape(1, -1), bn)


def kernel(x, edge_index, s1_t1w, s1_t1b, s1_chw, s1_chb, s1_t2w, s1_t2b, s1_bn,
           s2_t1w, s2_t1b, s2_chw, s2_chb, s2_t2w, s2_t2b, s2_bn,
           cls_w1, cls_b1, cls_w2, cls_b2):
    row = edge_index[0]
    col = edge_index[1]

    npad = NS * TSTEPS * GB - E
    rowp = jnp.concatenate([row, jnp.zeros((npad,), jnp.int32)]).reshape(NS * TSTEPS, GB)
    colp = jnp.concatenate([col, jnp.full((npad,), N, jnp.int32)]).reshape(NS * TSTEPS, GB)
    rowd = jnp.concatenate([row, jnp.full((npad,), N, jnp.int32)]).reshape(NS * TSTEPS, GB)
    zeros = jnp.zeros((ZSUB, HID), jnp.bfloat16)

    deg = _prop_sc(jnp.ones((NC, N, HID), jnp.bfloat16), rowp, rowd,
                   zeros)[0, :, 0:1].astype(jnp.float32)

    h1 = _stconv_block(x, rowp, colp, zeros, deg, s1_t1w, s1_t1b, s1_chw, s1_chb,
                       s1_t2w, s1_t2b, s1_bn, mean=False)
    hm = _stconv_block(h1, rowp, colp, zeros, deg, s2_t1w, s2_t1b, s2_chw, s2_chb,
                       s2_t2w, s2_t2b, s2_bn, mean=True)
    return _classifier_call(hm.reshape(B, N * OUT_CH), cls_w1, cls_b1, cls_w2, cls_b2)
